# baseline (speedup 1.0000x reference)
import functools

import numpy as np
import jax
import jax.numpy as jnp
from jax import lax
from jax.experimental import pallas as pl
from jax.experimental.pallas import tpu as pltpu
from jax.experimental.pallas import tpu_sc as plsc

NUM_HIDDENS = 1024
MAX_POS = 8192
B_TOTAL = 4 * 4096
SC_ROWS = 6144
TC_ROWS = B_TOTAL - SC_ROWS

NC = 2
NS = 16
NW = NC * NS
B_PER_W = SC_ROWS // NW
CHUNK = 32
NCHUNK = B_PER_W // CHUNK
NBUF = 3

TC_BLK = 1024
TC_BLK0 = SC_ROWS // TC_BLK


def _make_sc_gather():
    mesh = plsc.VectorSubcoreMesh(core_axis_name="c", subcore_axis_name="s")

    @functools.partial(
        pl.kernel,
        mesh=mesh,
        out_type=jax.ShapeDtypeStruct((SC_ROWS, NUM_HIDDENS), jnp.float32),
        scratch_types=[
            pltpu.VMEM((B_PER_W,), jnp.int32),
            pltpu.VMEM((NBUF, CHUNK, NUM_HIDDENS), jnp.float32),
        ]
        + [pltpu.SemaphoreType.DMA] * (2 * NBUF),
    )
    def k(table_hbm, idx_hbm, out_hbm, idx_v, rows_v, *sems):
        wid = lax.axis_index("s") * NC + lax.axis_index("c")
        base = wid * B_PER_W
        gsem = sems[:NBUF]
        ssem = sems[NBUF:]
        pltpu.sync_copy(idx_hbm.at[pl.ds(base, B_PER_W)], idx_v)

        def start_gather(g, b):
            return pltpu.async_copy(
                table_hbm.at[idx_v.at[pl.ds(g * CHUNK, CHUNK)]],
                rows_v.at[b],
                gsem[b],
            )

        def start_scatter(g, b):
            return pltpu.async_copy(
                rows_v.at[b],
                out_hbm.at[pl.ds(base + g * CHUNK, CHUNK)],
                ssem[b],
            )

        gather_h = [None] * NBUF
        scatter_h = [None] * NBUF
        gather_h[0] = start_gather(0, 0)
        for g in range(NCHUNK):
            b = g % NBUF
            if g + 1 < NCHUNK:
                nb = (g + 1) % NBUF
                if scatter_h[nb] is not None:
                    scatter_h[nb].wait()
                gather_h[nb] = start_gather(g + 1, nb)
            gather_h[b].wait()
            scatter_h[b] = start_scatter(g, b)
        for b in range(NBUF):
            if scatter_h[b] is not None:
                scatter_h[b].wait()

    return k


_sc_gather = _make_sc_gather()


def _make_tc_tables():
    j = np.arange(0, NUM_HIDDENS, 2, dtype=np.float64) / NUM_HIDDENS
    w = np.power(10000.0, -j)
    wc = np.repeat(w, 2)
    even = (np.arange(NUM_HIDDENS) % 2) == 0
    h = np.arange(MAX_POS // 64, dtype=np.float64).reshape(-1, 1)
    th_h = 64.0 * h * wc
    a1 = np.where(even, np.sin(th_h), np.cos(th_h))
    a2 = np.where(even, np.cos(th_h), -np.sin(th_h))
    l = np.arange(64, dtype=np.float64).reshape(-1, 1)
    th_l = l * wc
    b1 = np.cos(th_l)
    b2 = np.sin(th_l)
    A = np.concatenate([a1, a2], axis=1)
    B = np.concatenate([b1, b2], axis=1)
    return (
        jnp.asarray(A, dtype=jnp.bfloat16),
        jnp.asarray(B, dtype=jnp.bfloat16),
    )


_TC_A, _TC_B = _make_tc_tables()


def _tc_body(ids_ref, a_ref, b_ref, out_ref):
    ids = ids_ref[0]
    hi = ids >> 6
    lo = ids & 63
    row_h = lax.broadcasted_iota(jnp.int32, (MAX_POS // 64, TC_BLK), 0)
    row_l = lax.broadcasted_iota(jnp.int32, (64, TC_BLK), 0)
    ohT_hi = (hi == row_h).astype(jnp.bfloat16)
    ohT_lo = (lo == row_l).astype(jnp.bfloat16)
    dn = (((0,), (0,)), ((), ()))
    H = lax.dot_general(ohT_hi, a_ref[...], dn,
                        preferred_element_type=jnp.float32
                        ).astype(jnp.bfloat16)
    L = lax.dot_general(ohT_lo, b_ref[...], dn,
                        preferred_element_type=jnp.float32
                        ).astype(jnp.bfloat16)
    out_ref[...] = (
        H[:, :NUM_HIDDENS] * L[:, :NUM_HIDDENS]
        + H[:, NUM_HIDDENS:] * L[:, NUM_HIDDENS:]
    ).astype(jnp.float32)


def _tc_compute(ids3):
    nb = TC_ROWS // TC_BLK
    return pl.pallas_call(
        _tc_body,
        grid=(nb,),
        in_specs=[
            pl.BlockSpec((1, 1, TC_BLK), lambda i: (i + TC_BLK0, 0, 0)),
            pl.BlockSpec((MAX_POS // 64, 2 * NUM_HIDDENS), lambda i: (0, 0)),
            pl.BlockSpec((64, 2 * NUM_HIDDENS), lambda i: (0, 0)),
        ],
        out_specs=pl.BlockSpec((TC_BLK, NUM_HIDDENS), lambda i: (i + TC_BLK0, 0)),
        out_shape=jax.ShapeDtypeStruct((B_TOTAL, NUM_HIDDENS), jnp.float32),
    )(ids3, _TC_A, _TC_B)


def kernel(position_ids, PosEnc):
    ids = position_ids.reshape(-1).astype(jnp.int32)
    ids3 = ids.reshape(B_TOTAL // TC_BLK, 1, TC_BLK)
    sc_part = _sc_gather(PosEnc, ids)
    tc_out = _tc_compute(ids3)
    out = lax.dynamic_update_slice(tc_out, sc_part, (0, 0))
    return out.reshape(position_ids.shape + (NUM_HIDDENS,))


# speedup vs baseline: 1.1033x; 1.1033x over previous
import functools

import numpy as np
import jax
import jax.numpy as jnp
from jax import lax
from jax.experimental import pallas as pl
from jax.experimental.pallas import tpu as pltpu
from jax.experimental.pallas import tpu_sc as plsc

NUM_HIDDENS = 1024
MAX_POS = 8192
B_TOTAL = 4 * 4096
SC_ROWS = 2048
TC_ROWS = B_TOTAL - SC_ROWS

NC = 2
NS = 16
NW = NC * NS
B_PER_W = SC_ROWS // NW
CHUNK = 32
NCHUNK = B_PER_W // CHUNK
NBUF = 3

TC_BLK = 1024
TC_BLK0 = SC_ROWS // TC_BLK


def _make_sc_gather():
    mesh = plsc.VectorSubcoreMesh(core_axis_name="c", subcore_axis_name="s")

    @functools.partial(
        pl.kernel,
        mesh=mesh,
        out_type=jax.ShapeDtypeStruct((SC_ROWS, NUM_HIDDENS), jnp.float32),
        scratch_types=[
            pltpu.VMEM((B_PER_W,), jnp.int32),
            pltpu.VMEM((NBUF, CHUNK, NUM_HIDDENS), jnp.float32),
        ]
        + [pltpu.SemaphoreType.DMA] * (2 * NBUF),
    )
    def k(table_hbm, idx_hbm, out_hbm, idx_v, rows_v, *sems):
        wid = lax.axis_index("s") * NC + lax.axis_index("c")
        base = wid * B_PER_W
        gsem = sems[:NBUF]
        ssem = sems[NBUF:]
        pltpu.sync_copy(idx_hbm.at[pl.ds(base, B_PER_W)], idx_v)

        def start_gather(g, b):
            return pltpu.async_copy(
                table_hbm.at[idx_v.at[pl.ds(g * CHUNK, CHUNK)]],
                rows_v.at[b],
                gsem[b],
            )

        def start_scatter(g, b):
            return pltpu.async_copy(
                rows_v.at[b],
                out_hbm.at[pl.ds(base + g * CHUNK, CHUNK)],
                ssem[b],
            )

        gather_h = [None] * NBUF
        scatter_h = [None] * NBUF
        gather_h[0] = start_gather(0, 0)
        for g in range(NCHUNK):
            b = g % NBUF
            if g + 1 < NCHUNK:
                nb = (g + 1) % NBUF
                if scatter_h[nb] is not None:
                    scatter_h[nb].wait()
                gather_h[nb] = start_gather(g + 1, nb)
            gather_h[b].wait()
            scatter_h[b] = start_scatter(g, b)
        for b in range(NBUF):
            if scatter_h[b] is not None:
                scatter_h[b].wait()

    return k


_sc_gather = _make_sc_gather()


def _make_tc_tables():
    j = np.arange(0, NUM_HIDDENS, 2, dtype=np.float64) / NUM_HIDDENS
    w = np.power(10000.0, -j)
    wc = np.repeat(w, 2)
    even = (np.arange(NUM_HIDDENS) % 2) == 0
    h = np.arange(MAX_POS // 64, dtype=np.float64).reshape(-1, 1)
    th_h = 64.0 * h * wc
    a1 = np.where(even, np.sin(th_h), np.cos(th_h))
    a2 = np.where(even, np.cos(th_h), -np.sin(th_h))
    l = np.arange(64, dtype=np.float64).reshape(-1, 1)
    th_l = l * wc
    b1 = np.cos(th_l)
    b2 = np.sin(th_l)
    A = np.concatenate([a1, a2], axis=1)
    B = np.concatenate([b1, b2], axis=1)
    return (
        jnp.asarray(A, dtype=jnp.bfloat16),
        jnp.asarray(B, dtype=jnp.bfloat16),
    )


_TC_A, _TC_B = _make_tc_tables()


def _tc_body(ids_ref, a_ref, b_ref, out_ref):
    ids = ids_ref[0]
    hi = ids >> 6
    lo = ids & 63
    row_h = lax.broadcasted_iota(jnp.int32, (MAX_POS // 64, TC_BLK), 0)
    row_l = lax.broadcasted_iota(jnp.int32, (64, TC_BLK), 0)
    ohT_hi = (hi == row_h).astype(jnp.bfloat16)
    ohT_lo = (lo == row_l).astype(jnp.bfloat16)
    dn = (((0,), (0,)), ((), ()))
    H = lax.dot_general(ohT_hi, a_ref[...], dn,
                        preferred_element_type=jnp.float32
                        ).astype(jnp.bfloat16)
    L = lax.dot_general(ohT_lo, b_ref[...], dn,
                        preferred_element_type=jnp.float32
                        ).astype(jnp.bfloat16)
    out_ref[...] = (
        H[:, :NUM_HIDDENS] * L[:, :NUM_HIDDENS]
        + H[:, NUM_HIDDENS:] * L[:, NUM_HIDDENS:]
    ).astype(jnp.float32)


def _tc_compute(ids3):
    nb = TC_ROWS // TC_BLK
    return pl.pallas_call(
        _tc_body,
        grid=(nb,),
        in_specs=[
            pl.BlockSpec((1, 1, TC_BLK), lambda i: (i + TC_BLK0, 0, 0)),
            pl.BlockSpec((MAX_POS // 64, 2 * NUM_HIDDENS), lambda i: (0, 0)),
            pl.BlockSpec((64, 2 * NUM_HIDDENS), lambda i: (0, 0)),
        ],
        out_specs=pl.BlockSpec((TC_BLK, NUM_HIDDENS), lambda i: (i + TC_BLK0, 0)),
        out_shape=jax.ShapeDtypeStruct((B_TOTAL, NUM_HIDDENS), jnp.float32),
    )(ids3, _TC_A, _TC_B)


def kernel(position_ids, PosEnc):
    ids = position_ids.reshape(-1).astype(jnp.int32)
    ids3 = ids.reshape(B_TOTAL // TC_BLK, 1, TC_BLK)
    sc_part = _sc_gather(PosEnc, ids)
    tc_out = _tc_compute(ids3)
    out = lax.dynamic_update_slice(tc_out, sc_part, (0, 0))
    return out.reshape(position_ids.shape + (NUM_HIDDENS,))
